# baseline (speedup 1.0000x reference)
import jax
import jax.numpy as jnp
from jax.experimental import pallas as pl
from jax.experimental.pallas import tpu as pltpu

_B, _S, _D = 4, 8192, 768
_N = 16
_NG = 3
_KC, _KQK, _KV = 8, 4, 6
_TS = 8192


def _stream_body(x_ref, imp_ref, w_ref, g_ref, gt_ref, dw_ref, acc_ref):
    s = pl.program_id(1)
    ns = pl.num_programs(1)

    @pl.when(s == 0)
    def _init():
        acc_ref[...] = jnp.zeros_like(acc_ref)

    xt = x_ref[0]
    logits = jnp.dot(xt, w_ref[...], preferred_element_type=jnp.float32)
    e = jnp.exp(logits)
    d3 = jnp.dot(e, g_ref[...], preferred_element_type=jnp.float32)
    u = jnp.dot(1.0 / d3, gt_ref[...], preferred_element_type=jnp.float32)
    p = e * u
    imp = imp_ref[0, 0]
    acc_ref[...] += jnp.dot(imp, p, preferred_element_type=jnp.float32)

    @pl.when(s == ns - 1)
    def _finish():
        dw_ref[...] = acc_ref[...].reshape(1, 1, _NG * _N)


def _topk_group(w16, k, iota16):
    b = w16.shape[0]
    v = w16
    mask = jnp.zeros(w16.shape, dtype=jnp.bool_)
    iota_k = jax.lax.broadcasted_iota(jnp.int32, (b, k), 1)
    idx_rows = jnp.zeros((b, k), dtype=jnp.int32)
    for i in range(k):
        m = jnp.max(v, axis=-1, keepdims=True)
        ismax = v == m
        idx = jnp.min(jnp.where(ismax, iota16, _N), axis=-1, keepdims=True)
        sel = iota16 == idx
        mask = jnp.logical_or(mask, sel)
        v = jnp.where(sel, -1.0, v)
        idx_rows = jnp.where(iota_k == i, jnp.broadcast_to(idx, (b, k)), idx_rows)
    sparse = jnp.where(mask, w16, 0.0)
    sparse = sparse / (jnp.sum(sparse, axis=-1, keepdims=True) + 1e-8)
    return sparse, idx_rows


def _topk_body(dw_ref, cw_ref, qw_ref, vw_ref, ci_ref, qi_ref, vi_ref):
    w = dw_ref[...]
    iota16 = jax.lax.broadcasted_iota(jnp.int32, (_B, _N), 1)
    cw_ref[...], ci_ref[...] = _topk_group(w[:, 0:_N], _KC, iota16)
    qw_ref[...], qi_ref[...] = _topk_group(w[:, _N:2 * _N], _KQK, iota16)
    vw_ref[...], vi_ref[...] = _topk_group(w[:, 2 * _N:3 * _N], _KV, iota16)


def kernel(x, importance, Wc, Wqk, Wv):
    ns = _S // _TS
    w = jnp.concatenate([Wc, Wqk, Wv], axis=0).T
    imp = importance.reshape(_B, ns, 1, _TS)
    lanes = jnp.arange(_NG * _N)[:, None]
    g = (lanes // _N == jnp.arange(_NG)[None, :]).astype(jnp.float32)
    gt = g.T

    dense = pl.pallas_call(
        _stream_body,
        grid=(_B, ns),
        in_specs=[
            pl.BlockSpec((1, _TS, _D), lambda b, s: (b, s, 0)),
            pl.BlockSpec((1, 1, 1, _TS), lambda b, s: (b, s, 0, 0)),
            pl.BlockSpec((_D, _NG * _N), lambda b, s: (0, 0)),
            pl.BlockSpec((_NG * _N, _NG), lambda b, s: (0, 0)),
            pl.BlockSpec((_NG, _NG * _N), lambda b, s: (0, 0)),
        ],
        out_specs=pl.BlockSpec((1, 1, _NG * _N), lambda b, s: (b, 0, 0)),
        out_shape=jax.ShapeDtypeStruct((_B, 1, _NG * _N), jnp.float32),
        scratch_shapes=[pltpu.VMEM((1, _NG * _N), jnp.float32)],
        compiler_params=pltpu.CompilerParams(
            dimension_semantics=("parallel", "arbitrary"),
        ),
    )(x, imp, w, g, gt)

    out = pl.pallas_call(
        _topk_body,
        out_shape=[
            jax.ShapeDtypeStruct((_B, _N), jnp.float32),
            jax.ShapeDtypeStruct((_B, _N), jnp.float32),
            jax.ShapeDtypeStruct((_B, _N), jnp.float32),
            jax.ShapeDtypeStruct((_B, _KC), jnp.int32),
            jax.ShapeDtypeStruct((_B, _KQK), jnp.int32),
            jax.ShapeDtypeStruct((_B, _KV), jnp.int32),
        ],
    )(dense.reshape(_B, _NG * _N))

    return tuple(out)


# speedup vs baseline: 1.0197x; 1.0197x over previous
import jax
import jax.numpy as jnp
from jax.experimental import pallas as pl
from jax.experimental.pallas import tpu as pltpu

_B, _S, _D = 4, 8192, 768
_N = 16
_NG = 3
_KC, _KQK, _KV = 8, 4, 6
_TS = 4096


def _stream_body(x_ref, imp_ref, w_ref, g_ref, gt_ref, dw_ref, acc_ref):
    s = pl.program_id(1)
    ns = pl.num_programs(1)

    @pl.when(s == 0)
    def _init():
        acc_ref[...] = jnp.zeros_like(acc_ref)

    xt = x_ref[0].astype(jnp.bfloat16)
    logits = jnp.dot(xt, w_ref[...], preferred_element_type=jnp.float32)
    e = jnp.exp(logits)
    d3 = jnp.dot(e, g_ref[...], preferred_element_type=jnp.float32)
    u = jnp.dot(1.0 / d3, gt_ref[...], preferred_element_type=jnp.float32)
    p = e * u
    imp = imp_ref[0, 0]
    acc_ref[...] += jnp.dot(imp, p, preferred_element_type=jnp.float32)

    @pl.when(s == ns - 1)
    def _finish():
        dw_ref[...] = acc_ref[...].reshape(1, 1, _NG * _N)


def _topk_group(w16, k, iota16):
    b = w16.shape[0]
    v = w16
    mask = jnp.zeros(w16.shape, dtype=jnp.bool_)
    iota_k = jax.lax.broadcasted_iota(jnp.int32, (b, k), 1)
    idx_rows = jnp.zeros((b, k), dtype=jnp.int32)
    for i in range(k):
        m = jnp.max(v, axis=-1, keepdims=True)
        ismax = v == m
        idx = jnp.min(jnp.where(ismax, iota16, _N), axis=-1, keepdims=True)
        sel = iota16 == idx
        mask = jnp.logical_or(mask, sel)
        v = jnp.where(sel, -1.0, v)
        idx_rows = jnp.where(iota_k == i, jnp.broadcast_to(idx, (b, k)), idx_rows)
    sparse = jnp.where(mask, w16, 0.0)
    sparse = sparse / (jnp.sum(sparse, axis=-1, keepdims=True) + 1e-8)
    return sparse, idx_rows


def _topk_body(dw_ref, cw_ref, qw_ref, vw_ref, ci_ref, qi_ref, vi_ref):
    w = dw_ref[...]
    iota16 = jax.lax.broadcasted_iota(jnp.int32, (_B, _N), 1)
    cw_ref[...], ci_ref[...] = _topk_group(w[:, 0:_N], _KC, iota16)
    qw_ref[...], qi_ref[...] = _topk_group(w[:, _N:2 * _N], _KQK, iota16)
    vw_ref[...], vi_ref[...] = _topk_group(w[:, 2 * _N:3 * _N], _KV, iota16)


def kernel(x, importance, Wc, Wqk, Wv):
    ns = _S // _TS
    w = jnp.concatenate([Wc, Wqk, Wv], axis=0).T.astype(jnp.bfloat16)
    imp = importance.reshape(_B, ns, 1, _TS)
    lanes = jnp.arange(_NG * _N)[:, None]
    g = (lanes // _N == jnp.arange(_NG)[None, :]).astype(jnp.float32)
    gt = g.T

    dense = pl.pallas_call(
        _stream_body,
        grid=(_B, ns),
        in_specs=[
            pl.BlockSpec((1, _TS, _D), lambda b, s: (b, s, 0)),
            pl.BlockSpec((1, 1, 1, _TS), lambda b, s: (b, s, 0, 0)),
            pl.BlockSpec((_D, _NG * _N), lambda b, s: (0, 0)),
            pl.BlockSpec((_NG * _N, _NG), lambda b, s: (0, 0)),
            pl.BlockSpec((_NG, _NG * _N), lambda b, s: (0, 0)),
        ],
        out_specs=pl.BlockSpec((1, 1, _NG * _N), lambda b, s: (b, 0, 0)),
        out_shape=jax.ShapeDtypeStruct((_B, 1, _NG * _N), jnp.float32),
        scratch_shapes=[pltpu.VMEM((1, _NG * _N), jnp.float32)],
        compiler_params=pltpu.CompilerParams(
            dimension_semantics=("parallel", "arbitrary"),
        ),
    )(x, imp, w, g, gt)

    out = pl.pallas_call(
        _topk_body,
        out_shape=[
            jax.ShapeDtypeStruct((_B, _N), jnp.float32),
            jax.ShapeDtypeStruct((_B, _N), jnp.float32),
            jax.ShapeDtypeStruct((_B, _N), jnp.float32),
            jax.ShapeDtypeStruct((_B, _KC), jnp.int32),
            jax.ShapeDtypeStruct((_B, _KQK), jnp.int32),
            jax.ShapeDtypeStruct((_B, _KV), jnp.int32),
        ],
    )(dense.reshape(_B, _NG * _N))

    return tuple(out)


# speedup vs baseline: 1.1043x; 1.0830x over previous
import jax
import jax.numpy as jnp
from jax.experimental import pallas as pl
from jax.experimental.pallas import tpu as pltpu

_B, _S, _D = 4, 8192, 768
_N = 16
_NG = 3
_KC, _KQK, _KV = 8, 4, 6
_TS = 4096


def _stream_body(x_ref, imp_ref, w_ref, g48_ref, dw_ref, acc_ref):
    s = pl.program_id(1)
    ns = pl.num_programs(1)

    @pl.when(s == 0)
    def _init():
        acc_ref[...] = jnp.zeros_like(acc_ref)

    xt = x_ref[0]
    logits = jnp.dot(xt, w_ref[...], preferred_element_type=jnp.float32)
    e = jnp.exp(logits)
    d = jnp.dot(e, g48_ref[...], preferred_element_type=jnp.float32)
    p = e * (1.0 / d)
    imp = imp_ref[0, 0]
    acc_ref[...] += jnp.dot(imp, p, preferred_element_type=jnp.float32)

    @pl.when(s == ns - 1)
    def _finish():
        dw_ref[...] = acc_ref[...].reshape(1, 1, _NG * _N)


def _topk_group(w16, k, iota16):
    b = w16.shape[0]
    v = w16
    mask = jnp.zeros(w16.shape, dtype=jnp.bool_)
    iota_k = jax.lax.broadcasted_iota(jnp.int32, (b, k), 1)
    idx_rows = jnp.zeros((b, k), dtype=jnp.int32)
    for i in range(k):
        m = jnp.max(v, axis=-1, keepdims=True)
        ismax = v == m
        idx = jnp.min(jnp.where(ismax, iota16, _N), axis=-1, keepdims=True)
        sel = iota16 == idx
        mask = jnp.logical_or(mask, sel)
        v = jnp.where(sel, -1.0, v)
        idx_rows = jnp.where(iota_k == i, jnp.broadcast_to(idx, (b, k)), idx_rows)
    sparse = jnp.where(mask, w16, 0.0)
    sparse = sparse / (jnp.sum(sparse, axis=-1, keepdims=True) + 1e-8)
    return sparse, idx_rows


def _topk_body(dw_ref, cw_ref, qw_ref, vw_ref, ci_ref, qi_ref, vi_ref):
    w = dw_ref[...]
    iota16 = jax.lax.broadcasted_iota(jnp.int32, (_B, _N), 1)
    cw_ref[...], ci_ref[...] = _topk_group(w[:, 0:_N], _KC, iota16)
    qw_ref[...], qi_ref[...] = _topk_group(w[:, _N:2 * _N], _KQK, iota16)
    vw_ref[...], vi_ref[...] = _topk_group(w[:, 2 * _N:3 * _N], _KV, iota16)


def kernel(x, importance, Wc, Wqk, Wv):
    ns = _S // _TS
    w = jnp.concatenate([Wc, Wqk, Wv], axis=0).T
    imp = importance.reshape(_B, ns, 1, _TS)
    lanes = jnp.arange(_NG * _N)
    g48 = (lanes[:, None] // _N == lanes[None, :] // _N).astype(jnp.float32)

    dense = pl.pallas_call(
        _stream_body,
        grid=(_B, ns),
        in_specs=[
            pl.BlockSpec((1, _TS, _D), lambda b, s: (b, s, 0)),
            pl.BlockSpec((1, 1, 1, _TS), lambda b, s: (b, s, 0, 0)),
            pl.BlockSpec((_D, _NG * _N), lambda b, s: (0, 0)),
            pl.BlockSpec((_NG * _N, _NG * _N), lambda b, s: (0, 0)),
        ],
        out_specs=pl.BlockSpec((1, 1, _NG * _N), lambda b, s: (b, 0, 0)),
        out_shape=jax.ShapeDtypeStruct((_B, 1, _NG * _N), jnp.float32),
        scratch_shapes=[pltpu.VMEM((1, _NG * _N), jnp.float32)],
        compiler_params=pltpu.CompilerParams(
            dimension_semantics=("parallel", "arbitrary"),
        ),
    )(x, imp, w, g48)

    out = pl.pallas_call(
        _topk_body,
        out_shape=[
            jax.ShapeDtypeStruct((_B, _N), jnp.float32),
            jax.ShapeDtypeStruct((_B, _N), jnp.float32),
            jax.ShapeDtypeStruct((_B, _N), jnp.float32),
            jax.ShapeDtypeStruct((_B, _KC), jnp.int32),
            jax.ShapeDtypeStruct((_B, _KQK), jnp.int32),
            jax.ShapeDtypeStruct((_B, _KV), jnp.int32),
        ],
    )(dense.reshape(_B, _NG * _N))

    return tuple(out)
